# baseline (speedup 1.0000x reference)
import jax
import jax.numpy as jnp
from jax import lax
from jax.experimental import pallas as pl
from jax.experimental.pallas import tpu as pltpu
from jax.experimental.pallas import tpu_sc as plsc


def _scs_body(in_hbm, emb_hbm, out_hbm, sem):
    cid = lax.axis_index("c")
    pltpu.make_async_copy(
        emb_hbm, out_hbm.at[pl.ds(cid * 640, 640)], sem
    ).start()
    pltpu.make_async_copy(
        emb_hbm, out_hbm.at[pl.ds(cid * 640, 640)], sem
    ).wait()


@jax.jit
def kernel(inputs, emb_table):
    batch, rows, dim = inputs.shape
    n_emb = emb_table.shape[0]
    out_rows = rows + n_emb
    mesh = plsc.ScalarSubcoreMesh(axis_name="c", num_cores=2)
    run = pl.kernel(
        _scs_body,
        out_type=jax.ShapeDtypeStruct((batch * out_rows * dim,), inputs.dtype),
        mesh=mesh,
        scratch_types=[
            pltpu.SemaphoreType.DMA,
        ],
    )
    flat = run(inputs.reshape(-1), emb_table.reshape(-1))
    return flat.reshape(batch, out_rows, dim)


# speedup vs baseline: 1.0471x; 1.0471x over previous
import jax
import jax.numpy as jnp
from jax import lax
from jax.experimental import pallas as pl
from jax.experimental.pallas import tpu as pltpu
from jax.experimental.pallas import tpu_sc as plsc

_NC = 2
_NS = 16
_NW = _NC * _NS
_NBUF = 4


def _sc_body(in_hbm, emb_hbm, out_hbm, bufs, esem, in_sems, out_sems):
    batch, rows, dim = in_hbm.shape
    n_emb = emb_hbm.shape[0]
    per_w = batch // _NW

    wid = lax.axis_index("s") * _NC + lax.axis_index("c")
    base = wid * per_w

    def in_copy(g, i):
        return pltpu.make_async_copy(
            in_hbm.at[base + g],
            bufs[i].at[pl.ds(n_emb, rows)],
            in_sems[i],
        )

    def out_copy(g, i):
        return pltpu.make_async_copy(
            bufs[i],
            out_hbm.at[base + g],
            out_sems[i],
        )

    def emb_copy(i):
        return pltpu.make_async_copy(
            emb_hbm, bufs[i].at[pl.ds(0, n_emb)], esem
        )

    for g in range(min(_NBUF - 1, per_w)):
        in_copy(g, g).start()
    for i in range(_NBUF):
        emb_copy(i).start()
    for i in range(_NBUF):
        emb_copy(i).wait()

    for g in range(per_w):
        i = g % _NBUF
        in_copy(g, i).wait()
        out_copy(g, i).start()
        nxt = g + _NBUF - 1
        if nxt < per_w:
            if g >= 1:
                out_copy(g - 1, (g - 1) % _NBUF).wait()
            in_copy(nxt, nxt % _NBUF).start()
    out_copy(per_w - 1, (per_w - 1) % _NBUF).wait()


@jax.jit
def kernel(inputs, emb_table):
    batch, rows, dim = inputs.shape
    n_emb = emb_table.shape[0]
    out_rows = rows + n_emb
    mesh = plsc.VectorSubcoreMesh(core_axis_name="c", subcore_axis_name="s")
    run = pl.kernel(
        _sc_body,
        out_type=jax.ShapeDtypeStruct((batch, out_rows, dim), inputs.dtype),
        mesh=mesh,
        scratch_types=[
            [pltpu.VMEM((out_rows, dim), inputs.dtype) for _ in range(_NBUF)],
            pltpu.SemaphoreType.DMA,
            [pltpu.SemaphoreType.DMA for _ in range(_NBUF)],
            [pltpu.SemaphoreType.DMA for _ in range(_NBUF)],
        ],
    )
    return run(inputs, emb_table)
